# Initial kernel scaffold; baseline (speedup 1.0000x reference)
import jax
import jax.numpy as jnp
from jax.experimental import pallas as pl


def kernel(inputs, routing_x):
    raise NotImplementedError("write your pallas kernel here")


# speedup vs baseline: 1.6603x; 1.6603x over previous
import functools

import jax
import jax.numpy as jnp
from jax import lax
from jax.experimental import pallas as pl
from jax.experimental.pallas import tpu as pltpu
from jax.experimental.pallas import tpu_sc as plsc


def _routed_gather(rows_total, width, routes, num_batches):
    info = plsc.get_sparse_core_info()
    nc, ns, lanes = info.num_cores, info.num_subcores, info.num_lanes
    nw = nc * ns
    assert rows_total % nw == 0
    rows_per_w = rows_total // nw
    rows_per_batch = rows_total // num_batches
    assert rows_per_batch % rows_per_w == 0
    w_per_batch = rows_per_batch // rows_per_w
    n_chunks = max(1, rows_per_w // 128)
    chunk = rows_per_w // n_chunks
    assert chunk % lanes == 0

    mesh = plsc.VectorSubcoreMesh(core_axis_name="c", subcore_axis_name="s")

    @functools.partial(
        pl.kernel,
        mesh=mesh,
        compiler_params=pltpu.CompilerParams(needs_layout_passes=False,
                                             use_tc_tiling_on_sc=False),
        out_type=jax.ShapeDtypeStruct((rows_total, width), jnp.float32),
        scratch_types=[
            pltpu.VMEM((lanes,), jnp.float32),
            pltpu.VMEM((n_chunks, chunk), jnp.int32),
            pltpu.VMEM((rows_per_w, width), jnp.float32),
            pltpu.SemaphoreType.DMA,
        ],
    )
    def k(table_hbm, routing_hbm, out_hbm, routing_v, idx_v, rows_v, sem):
        wid = lax.axis_index("s") * nc + lax.axis_index("c")
        base_row = wid * rows_per_w
        b = wid // w_per_batch

        pltpu.sync_copy(routing_hbm.at[pl.ds(b * lanes, lanes)], routing_v)
        v = routing_v[...]
        l_ids = lax.iota(jnp.int32, lanes)
        mx = plsc.load_gather(routing_v, [jnp.zeros((lanes,), jnp.int32)])
        for r in range(1, routes):
            mx = jnp.maximum(
                mx, plsc.load_gather(routing_v, [jnp.full((lanes,), r, jnp.int32)]))
        route = plsc.all_reduce_ffs((v == mx) & (l_ids < routes))

        for j in range(n_chunks):
            for c in range(chunk // lanes):
                r16 = (base_row + j * chunk + c * lanes) + l_ids
                idx_v[j, pl.ds(c * lanes, lanes)] = r16 * routes + route

        copies = [
            pltpu.async_copy(table_hbm.at[idx_v.at[j]],
                             rows_v.at[pl.ds(j * chunk, chunk)], sem)
            for j in range(n_chunks)
        ]
        for cp in copies:
            cp.wait()
        pltpu.sync_copy(rows_v, out_hbm.at[pl.ds(base_row, rows_per_w)])

    return k


def kernel(inputs, routing_x):
    bsz, h, w_sp, c = inputs.shape
    routes = routing_x.shape[-1]
    width = c // routes
    rows_total = bsz * h * w_sp
    table = inputs.reshape(rows_total * routes, width)
    routing_pad = jnp.pad(routing_x, ((0, 0), (0, 16 - routes)),
                          constant_values=-jnp.inf).reshape(-1)
    out = _routed_gather(rows_total, width, routes, bsz)(table, routing_pad)
    return out.reshape(bsz, h, w_sp, width)
